# Initial kernel scaffold; baseline (speedup 1.0000x reference)
import jax
import jax.numpy as jnp
from jax.experimental import pallas as pl


def kernel(user_sup_indices, user_sup_values, item_sup_indices, item_sup_values, user_inputs, item_inputs, weight):
    raise NotImplementedError("write your pallas kernel here")


# speedup vs baseline: 3.1362x; 3.1362x over previous
import functools

import jax
import jax.numpy as jnp
from jax import lax
from jax.experimental import pallas as pl
from jax.experimental.pallas import tpu as pltpu
from jax.experimental.pallas import tpu_sc as plsc

N_NODES = 10000
N_SUP = 5
E_EDGES = 100000
D_IN = 128
D_OUT = 64
N_STACK = 2 * N_NODES

NUM_TILES = 16
CHUNK = 128
EP = 100352
EPT = EP // NUM_TILES
NCH = EPT // CHUNK
N_PAD = 10240
ROWS_PER_TILE = N_PAD // NUM_TILES


def _bcast_lane(vv, j):
    idx = jnp.full((16, 1), j, jnp.int32)
    dn = lax.GatherDimensionNumbers(
        offset_dims=(), collapsed_slice_dims=(0,), start_index_map=(0,))
    return lax.gather(vv, idx, dn, (1,),
                      mode=lax.GatherScatterMode.PROMISE_IN_BOUNDS)


def _mm_body(x_ref, w_ref, o_ref):
    o_ref[0] = jnp.dot(x_ref[...], w_ref[0], preferred_element_type=jnp.float32)


def _dense_projections(stacked, weight):
    BM = 2000
    return pl.pallas_call(
        _mm_body,
        grid=(N_SUP, N_STACK // BM),
        in_specs=[
            pl.BlockSpec((BM, D_IN), lambda i, j: (j, 0)),
            pl.BlockSpec((1, D_IN, D_OUT), lambda i, j: (i, 0, 0)),
        ],
        out_specs=pl.BlockSpec((1, BM, D_OUT), lambda i, j: (i, j, 0)),
        out_shape=jax.ShapeDtypeStruct((N_SUP, N_STACK, D_OUT), jnp.float32),
    )(stacked, weight)


def _sc_aggregate(table_flat, src_all, dst_all, val_all):
    mesh = plsc.VectorSubcoreMesh(core_axis_name="c", subcore_axis_name="s")

    @functools.partial(
        pl.kernel,
        mesh=mesh,
        compiler_params=pltpu.CompilerParams(use_tc_tiling_on_sc=False),
        out_type=jax.ShapeDtypeStruct((2, N_SUP, N_PAD, D_OUT), jnp.float32),
        scratch_types=[
            pltpu.VMEM((CHUNK,), jnp.int32),
            pltpu.VMEM((CHUNK,), jnp.int32),
            pltpu.VMEM((CHUNK,), jnp.float32),
            pltpu.VMEM((CHUNK, D_OUT), jnp.float32),
            pltpu.VMEM_SHARED((N_PAD, D_OUT), jnp.float32),
            pltpu.SemaphoreType.DMA,
        ],
    )
    def k(table_hbm, src_hbm, dst_hbm, val_hbm, out_hbm,
          srci_v, dsti_v, vals_v, rows_v, acc_sh, sem):
        c = lax.axis_index("c")
        s = lax.axis_index("s")
        my_rows = s * ROWS_PER_TILE
        RB = ROWS_PER_TILE // CHUNK

        def pass_body(q, _):
            p = c * N_SUP + q
            src_off = q * N_STACK + (1 - c) * N_NODES
            base = p * EP + s * EPT

            def z_body(r, _):
                for kk in range(D_OUT // 16):
                    rows_v[r, pl.ds(kk * 16, 16)] = jnp.zeros((16,), jnp.float32)
                return 0
            lax.fori_loop(0, CHUNK, z_body, 0)

            def zc_body(b, _):
                pltpu.sync_copy(rows_v, acc_sh.at[pl.ds(my_rows + b * CHUNK, CHUNK)])
                return 0
            lax.fori_loop(0, RB, zc_body, 0)
            plsc.subcore_barrier()

            def chunk_body(t, _):
                eb = base + t * CHUNK
                pltpu.sync_copy(src_hbm.at[pl.ds(eb, CHUNK)], srci_v)
                pltpu.sync_copy(dst_hbm.at[pl.ds(eb, CHUNK)], dsti_v)
                pltpu.sync_copy(val_hbm.at[pl.ds(eb, CHUNK)], vals_v)

                def off_body(g, _):
                    srci_v[pl.ds(g * 16, 16)] = srci_v[pl.ds(g * 16, 16)] + src_off
                    return 0
                lax.fori_loop(0, CHUNK // 16, off_body, 0)

                pltpu.async_copy(table_hbm.at[srci_v], rows_v, sem).wait()

                def scale_body(g, _):
                    vv = vals_v[pl.ds(g * 16, 16)]
                    for j in range(16):
                        sv = _bcast_lane(vv, j)
                        e = g * 16 + j
                        for kk in range(D_OUT // 16):
                            rows_v[e, pl.ds(kk * 16, 16)] = (
                                rows_v[e, pl.ds(kk * 16, 16)] * sv)
                    return 0
                lax.fori_loop(0, CHUNK // 16, scale_body, 0)

                pltpu.sync_copy(rows_v, acc_sh.at[dsti_v], add=True)
                return 0
            lax.fori_loop(0, NCH, chunk_body, 0)
            plsc.subcore_barrier()

            def out_body(b, _):
                rb = my_rows + b * CHUNK
                pltpu.sync_copy(acc_sh.at[pl.ds(rb, CHUNK)], rows_v)

                def relu_body(r, _):
                    for kk in range(D_OUT // 16):
                        v = rows_v[r, pl.ds(kk * 16, 16)]
                        rows_v[r, pl.ds(kk * 16, 16)] = jnp.maximum(v, 0.0)
                    return 0
                lax.fori_loop(0, CHUNK, relu_body, 0)

                pltpu.sync_copy(rows_v, out_hbm.at[c, q, pl.ds(rb, CHUNK)])
                return 0
            lax.fori_loop(0, RB, out_body, 0)
            plsc.subcore_barrier()
            return 0
        lax.fori_loop(0, N_SUP, pass_body, 0)

    return k(table_flat, src_all, dst_all, val_all)


def kernel(user_sup_indices, user_sup_values, item_sup_indices, item_sup_values,
           user_inputs, item_inputs, weight):
    stacked = jnp.concatenate([user_inputs, item_inputs], axis=0)
    table = _dense_projections(stacked, weight).reshape(N_SUP * N_STACK, D_OUT)

    pad = ((0, 0), (0, EP - E_EDGES))
    u_src = jnp.pad(user_sup_indices[:, 1, :], pad)
    u_dst = jnp.pad(user_sup_indices[:, 0, :], pad)
    u_val = jnp.pad(user_sup_values, pad)
    i_src = jnp.pad(item_sup_indices[:, 1, :], pad)
    i_dst = jnp.pad(item_sup_indices[:, 0, :], pad)
    i_val = jnp.pad(item_sup_values, pad)
    src_all = jnp.concatenate([u_src, i_src]).reshape(-1)
    dst_all = jnp.concatenate([u_dst, i_dst]).reshape(-1)
    val_all = jnp.concatenate([u_val, i_val]).reshape(-1)

    out = _sc_aggregate(table, src_all, dst_all, val_all)
    user_out = out[0, :, :N_NODES].transpose(1, 0, 2).reshape(N_NODES, N_SUP * D_OUT)
    item_out = out[1, :, :N_NODES].transpose(1, 0, 2).reshape(N_NODES, N_SUP * D_OUT)
    return (user_out, item_out)


# speedup vs baseline: 4.0257x; 1.2836x over previous
import functools

import jax
import jax.numpy as jnp
from jax import lax
from jax.experimental import pallas as pl
from jax.experimental.pallas import tpu as pltpu
from jax.experimental.pallas import tpu_sc as plsc

N_NODES = 10000
N_SUP = 5
E_EDGES = 100000
D_IN = 128
D_OUT = 64
N_STACK = 2 * N_NODES

NUM_TILES = 16
CHUNK = 128
EP = 102400
EPT = EP // NUM_TILES
NCH = EPT // CHUNK
N_PAD = 10240
ROWS_PER_TILE = N_PAD // NUM_TILES


def _bcast_lane(vv, j):
    idx = jnp.full((16, 1), j, jnp.int32)
    dn = lax.GatherDimensionNumbers(
        offset_dims=(), collapsed_slice_dims=(0,), start_index_map=(0,))
    return lax.gather(vv, idx, dn, (1,),
                      mode=lax.GatherScatterMode.PROMISE_IN_BOUNDS)


def _mm_body(x_ref, w_ref, o_ref):
    o_ref[0] = jnp.dot(x_ref[...], w_ref[0], preferred_element_type=jnp.float32)


def _dense_projections(stacked, weight):
    BM = 2000
    return pl.pallas_call(
        _mm_body,
        grid=(N_SUP, N_STACK // BM),
        in_specs=[
            pl.BlockSpec((BM, D_IN), lambda i, j: (j, 0)),
            pl.BlockSpec((1, D_IN, D_OUT), lambda i, j: (i, 0, 0)),
        ],
        out_specs=pl.BlockSpec((1, BM, D_OUT), lambda i, j: (i, j, 0)),
        out_shape=jax.ShapeDtypeStruct((N_SUP, N_STACK, D_OUT), jnp.float32),
    )(stacked, weight)


def _sc_aggregate(table_flat, src_all, dst_all, val_all):
    mesh = plsc.VectorSubcoreMesh(core_axis_name="c", subcore_axis_name="s")

    @functools.partial(
        pl.kernel,
        mesh=mesh,
        compiler_params=pltpu.CompilerParams(use_tc_tiling_on_sc=False),
        out_type=jax.ShapeDtypeStruct((2, N_SUP, N_PAD, D_OUT), jnp.float32),
        scratch_types=[
            pltpu.VMEM((EPT,), jnp.int32),
            pltpu.VMEM((EPT,), jnp.int32),
            pltpu.VMEM((EPT,), jnp.float32),
            pltpu.VMEM((CHUNK, D_OUT), jnp.float32),
            pltpu.VMEM((CHUNK, D_OUT), jnp.float32),
            pltpu.VMEM((CHUNK,), jnp.int32),
            pltpu.VMEM((CHUNK,), jnp.int32),
            pltpu.VMEM((CHUNK,), jnp.int32),
            pltpu.VMEM((CHUNK,), jnp.int32),
            pltpu.VMEM_SHARED((N_PAD, D_OUT), jnp.float32),
            pltpu.SemaphoreType.DMA,
            pltpu.SemaphoreType.DMA,
            pltpu.SemaphoreType.DMA,
            pltpu.SemaphoreType.DMA,
            pltpu.SemaphoreType.DMA,
        ],
    )
    def k(table_hbm, src_hbm, dst_hbm, val_hbm, out_hbm,
          srci_v, dsti_v, vals_v, rows0, rows1,
          srcc0, srcc1, dstc0, dstc1, acc_sh,
          sem_e, sem_g0, sem_g1, sem_s0, sem_s1):
        c = lax.axis_index("c")
        s = lax.axis_index("s")
        my_rows = s * ROWS_PER_TILE
        RB = ROWS_PER_TILE // CHUNK
        bufs = (rows0, rows1)
        srccs = (srcc0, srcc1)
        dstcs = (dstc0, dstc1)
        gsems = (sem_g0, sem_g1)
        ssems = (sem_s0, sem_s1)

        def fill_src(t, b, src_off):
            for g in range(CHUNK // 16):
                srccs[b][pl.ds(g * 16, 16)] = (
                    srci_v[pl.ds(t * CHUNK + g * 16, 16)] + src_off)

        def fill_dst(t, b):
            for g in range(CHUNK // 16):
                dstcs[b][pl.ds(g * 16, 16)] = dsti_v[pl.ds(t * CHUNK + g * 16, 16)]

        def scale(rows_v, t, g, _):
            vv = vals_v[pl.ds(t * CHUNK + g * 16, 16)]
            for j in range(16):
                sv = _bcast_lane(vv, j)
                e = g * 16 + j
                for kk in range(D_OUT // 16):
                    rows_v[e, pl.ds(kk * 16, 16)] = (
                        rows_v[e, pl.ds(kk * 16, 16)] * sv)
            return 0

        def gather_copy(b):
            return pltpu.make_async_copy(
                table_hbm.at[srccs[b]], bufs[b], gsems[b])

        def scatter_copy(b):
            return pltpu.make_async_copy(
                bufs[b], acc_sh.at[dstcs[b]], ssems[b])

        def pass_body(q, _):
            p = c * N_SUP + q
            src_off = q * N_STACK + (1 - c) * N_NODES

            ebase = p * EP + s * EPT
            e0 = pltpu.make_async_copy(src_hbm.at[pl.ds(ebase, EPT)], srci_v, sem_e)
            e1 = pltpu.make_async_copy(dst_hbm.at[pl.ds(ebase, EPT)], dsti_v, sem_e)
            e2 = pltpu.make_async_copy(val_hbm.at[pl.ds(ebase, EPT)], vals_v, sem_e)
            e0.start()
            e1.start()
            e2.start()

            def z_body(r, _):
                for kk in range(D_OUT // 16):
                    rows0[r, pl.ds(kk * 16, 16)] = jnp.zeros((16,), jnp.float32)
                return 0
            lax.fori_loop(0, CHUNK, z_body, 0)

            def zc_body(b, _):
                pltpu.sync_copy(rows0, acc_sh.at[pl.ds(my_rows + b * CHUNK, CHUNK)])
                return 0
            lax.fori_loop(0, RB, zc_body, 0)

            e0.wait()
            e1.wait()
            e2.wait()
            plsc.subcore_barrier()

            for b in range(2):
                fill_src(b, b, src_off)
                gather_copy(b).start()

            def chunk_body(t, _):
                for b in range(2):
                    tb = t + b
                    gather_copy(b).wait()
                    lax.fori_loop(0, CHUNK // 16,
                                  functools.partial(scale, bufs[b], tb), 0)
                    fill_dst(tb, b)
                    scatter_copy(b).start(add=True)
                for b in range(2):
                    tb = t + b
                    scatter_copy(b).wait()

                    @pl.when(tb + 2 < NCH)
                    def _():
                        fill_src(tb + 2, b, src_off)
                        gather_copy(b).start()
                return 0
            lax.fori_loop(0, NCH // 2, lambda i, u: chunk_body(i * 2, u), 0)
            plsc.subcore_barrier()

            def out_body(b, _):
                rb = my_rows + b * CHUNK
                pltpu.sync_copy(acc_sh.at[pl.ds(rb, CHUNK)], rows0)

                def relu_body(r, _):
                    for kk in range(D_OUT // 16):
                        v = rows0[r, pl.ds(kk * 16, 16)]
                        rows0[r, pl.ds(kk * 16, 16)] = jnp.maximum(v, 0.0)
                    return 0
                lax.fori_loop(0, CHUNK, relu_body, 0)

                pltpu.sync_copy(rows0, out_hbm.at[c, q, pl.ds(rb, CHUNK)])
                return 0
            lax.fori_loop(0, RB, out_body, 0)
            plsc.subcore_barrier()
            return 0
        lax.fori_loop(0, N_SUP, pass_body, 0)

    return k(table_flat, src_all, dst_all, val_all)


def kernel(user_sup_indices, user_sup_values, item_sup_indices, item_sup_values,
           user_inputs, item_inputs, weight):
    stacked = jnp.concatenate([user_inputs, item_inputs], axis=0)
    table = _dense_projections(stacked, weight).reshape(N_SUP * N_STACK, D_OUT)

    pad = ((0, 0), (0, EP - E_EDGES))
    u_src = jnp.pad(user_sup_indices[:, 1, :], pad)
    u_dst = jnp.pad(user_sup_indices[:, 0, :], pad)
    u_val = jnp.pad(user_sup_values, pad)
    i_src = jnp.pad(item_sup_indices[:, 1, :], pad)
    i_dst = jnp.pad(item_sup_indices[:, 0, :], pad)
    i_val = jnp.pad(item_sup_values, pad)
    src_all = jnp.concatenate([u_src, i_src]).reshape(-1)
    dst_all = jnp.concatenate([u_dst, i_dst]).reshape(-1)
    val_all = jnp.concatenate([u_val, i_val]).reshape(-1)

    out = _sc_aggregate(table, src_all, dst_all, val_all)
    user_out = out[0, :, :N_NODES].transpose(1, 0, 2).reshape(N_NODES, N_SUP * D_OUT)
    item_out = out[1, :, :N_NODES].transpose(1, 0, 2).reshape(N_NODES, N_SUP * D_OUT)
    return (user_out, item_out)


# speedup vs baseline: 5.6017x; 1.3915x over previous
import functools

import jax
import jax.numpy as jnp
from jax import lax
from jax.experimental import pallas as pl
from jax.experimental.pallas import tpu as pltpu
from jax.experimental.pallas import tpu_sc as plsc

N_NODES = 10000
N_SUP = 5
E_EDGES = 100000
D_IN = 128
D_OUT = 64
N_STACK = 2 * N_NODES

NUM_TILES = 16
CHUNK = 128
EP = 102400
EPT = EP // NUM_TILES
NCH = EPT // CHUNK
N_PAD = 10240
ROWS_PER_TILE = N_PAD // NUM_TILES


def _bcast_lane(vv, j):
    idx = jnp.full((16, 1), j, jnp.int32)
    dn = lax.GatherDimensionNumbers(
        offset_dims=(), collapsed_slice_dims=(0,), start_index_map=(0,))
    return lax.gather(vv, idx, dn, (1,),
                      mode=lax.GatherScatterMode.PROMISE_IN_BOUNDS)


def _mm_body(x_ref, w_ref, o_ref):
    o_ref[0] = jnp.dot(x_ref[...], w_ref[0], preferred_element_type=jnp.float32)


def _dense_projections(stacked, weight):
    BM = 2000
    return pl.pallas_call(
        _mm_body,
        grid=(N_SUP, N_STACK // BM),
        in_specs=[
            pl.BlockSpec((BM, D_IN), lambda i, j: (j, 0)),
            pl.BlockSpec((1, D_IN, D_OUT), lambda i, j: (i, 0, 0)),
        ],
        out_specs=pl.BlockSpec((1, BM, D_OUT), lambda i, j: (i, j, 0)),
        out_shape=jax.ShapeDtypeStruct((N_SUP, N_STACK, D_OUT), jnp.float32),
    )(stacked, weight)


def _sc_aggregate(table_flat, src_all, dst_all, val_all):
    mesh = plsc.VectorSubcoreMesh(core_axis_name="c", subcore_axis_name="s")

    @functools.partial(
        pl.kernel,
        mesh=mesh,
        compiler_params=pltpu.CompilerParams(use_tc_tiling_on_sc=False),
        out_type=jax.ShapeDtypeStruct((2, N_SUP, N_PAD, D_OUT), jnp.float32),
        scratch_types=[
            pltpu.VMEM((EPT,), jnp.int32),
            pltpu.VMEM((EPT,), jnp.int32),
            pltpu.VMEM((EPT,), jnp.float32),
            pltpu.VMEM((CHUNK, D_OUT), jnp.float32),
            pltpu.VMEM((CHUNK, D_OUT), jnp.float32),
            pltpu.VMEM((CHUNK,), jnp.int32),
            pltpu.VMEM((CHUNK,), jnp.int32),
            pltpu.VMEM((CHUNK,), jnp.int32),
            pltpu.VMEM((CHUNK,), jnp.int32),
            pltpu.VMEM_SHARED((N_PAD, D_OUT), jnp.float32),
            pltpu.SemaphoreType.DMA,
            pltpu.SemaphoreType.DMA,
            pltpu.SemaphoreType.DMA,
            pltpu.SemaphoreType.DMA,
            pltpu.SemaphoreType.DMA,
        ],
    )
    def k(table_hbm, src_hbm, dst_hbm, val_hbm, out_hbm,
          srci_v, dsti_v, vals_v, rows0, rows1,
          srcc0, srcc1, dstc0, dstc1, acc_sh,
          sem_e, sem_g0, sem_g1, sem_s0, sem_s1):
        c = lax.axis_index("c")
        s = lax.axis_index("s")
        my_rows = s * ROWS_PER_TILE
        RB = ROWS_PER_TILE // CHUNK
        bufs = (rows0, rows1)
        srccs = (srcc0, srcc1)
        dstcs = (dstc0, dstc1)
        gsems = (sem_g0, sem_g1)
        ssems = (sem_s0, sem_s1)

        def fill_src(t, b, src_off):
            blocks = [srci_v[pl.ds(t * CHUNK + g * 16, 16)]
                      for g in range(CHUNK // 16)]
            for g in range(CHUNK // 16):
                srccs[b][pl.ds(g * 16, 16)] = blocks[g] + src_off

        def fill_dst(t, b):
            blocks = [dsti_v[pl.ds(t * CHUNK + g * 16, 16)]
                      for g in range(CHUNK // 16)]
            for g in range(CHUNK // 16):
                dstcs[b][pl.ds(g * 16, 16)] = blocks[g]

        def scale_chunk(rows_v, t):
            @plsc.parallel_loop(0, CHUNK // 16)
            def _(g):
                vv = vals_v[pl.ds(t * CHUNK + g * 16, 16)]
                svs = [_bcast_lane(vv, j) for j in range(16)]
                for j in range(16):
                    e = g * 16 + j
                    blocks = [rows_v[e, pl.ds(kk * 16, 16)]
                              for kk in range(D_OUT // 16)]
                    for kk in range(D_OUT // 16):
                        rows_v[e, pl.ds(kk * 16, 16)] = blocks[kk] * svs[j]

        def gather_copy(b):
            return pltpu.make_async_copy(
                table_hbm.at[srccs[b]], bufs[b], gsems[b])

        def scatter_copy(b):
            return pltpu.make_async_copy(
                bufs[b], acc_sh.at[dstcs[b]], ssems[b])

        def pass_body(q, _):
            p = c * N_SUP + q
            src_off = q * N_STACK + (1 - c) * N_NODES

            ebase = p * EP + s * EPT
            e0 = pltpu.make_async_copy(src_hbm.at[pl.ds(ebase, EPT)], srci_v, sem_e)
            e1 = pltpu.make_async_copy(dst_hbm.at[pl.ds(ebase, EPT)], dsti_v, sem_e)
            e2 = pltpu.make_async_copy(val_hbm.at[pl.ds(ebase, EPT)], vals_v, sem_e)
            e0.start()
            e1.start()
            e2.start()

            @plsc.parallel_loop(0, CHUNK)
            def _(r):
                for kk in range(D_OUT // 16):
                    rows0[r, pl.ds(kk * 16, 16)] = jnp.zeros((16,), jnp.float32)

            def zc_body(b, _):
                pltpu.sync_copy(rows0, acc_sh.at[pl.ds(my_rows + b * CHUNK, CHUNK)])
                return 0
            lax.fori_loop(0, RB, zc_body, 0)

            e0.wait()
            e1.wait()
            e2.wait()
            plsc.subcore_barrier()

            for b in range(2):
                fill_src(b, b, src_off)
                gather_copy(b).start()

            def chunk_body(t, _):
                for b in range(2):
                    tb = t + b
                    gather_copy(b).wait()
                    scale_chunk(bufs[b], tb)
                    fill_dst(tb, b)
                    scatter_copy(b).start(add=True)
                for b in range(2):
                    tb = t + b
                    scatter_copy(b).wait()

                    @pl.when(tb + 2 < NCH)
                    def _():
                        fill_src(tb + 2, b, src_off)
                        gather_copy(b).start()
                return 0
            lax.fori_loop(0, NCH // 2, lambda i, u: chunk_body(i * 2, u), 0)
            plsc.subcore_barrier()

            def out_body(b, _):
                rb = my_rows + b * CHUNK
                pltpu.sync_copy(acc_sh.at[pl.ds(rb, CHUNK)], rows0)

                @plsc.parallel_loop(0, CHUNK)
                def _(r):
                    for kk in range(D_OUT // 16):
                        v = rows0[r, pl.ds(kk * 16, 16)]
                        rows0[r, pl.ds(kk * 16, 16)] = jnp.maximum(v, 0.0)

                pltpu.sync_copy(rows0, out_hbm.at[c, q, pl.ds(rb, CHUNK)])
                return 0
            lax.fori_loop(0, RB, out_body, 0)
            plsc.subcore_barrier()
            return 0
        lax.fori_loop(0, N_SUP, pass_body, 0)

    return k(table_flat, src_all, dst_all, val_all)


def kernel(user_sup_indices, user_sup_values, item_sup_indices, item_sup_values,
           user_inputs, item_inputs, weight):
    stacked = jnp.concatenate([user_inputs, item_inputs], axis=0)
    table = _dense_projections(stacked, weight).reshape(N_SUP * N_STACK, D_OUT)

    pad = ((0, 0), (0, EP - E_EDGES))
    u_src = jnp.pad(user_sup_indices[:, 1, :], pad)
    u_dst = jnp.pad(user_sup_indices[:, 0, :], pad)
    u_val = jnp.pad(user_sup_values, pad)
    i_src = jnp.pad(item_sup_indices[:, 1, :], pad)
    i_dst = jnp.pad(item_sup_indices[:, 0, :], pad)
    i_val = jnp.pad(item_sup_values, pad)
    src_all = jnp.concatenate([u_src, i_src]).reshape(-1)
    dst_all = jnp.concatenate([u_dst, i_dst]).reshape(-1)
    val_all = jnp.concatenate([u_val, i_val]).reshape(-1)

    out = _sc_aggregate(table, src_all, dst_all, val_all)
    user_out = out[0, :, :N_NODES].transpose(1, 0, 2).reshape(N_NODES, N_SUP * D_OUT)
    item_out = out[1, :, :N_NODES].transpose(1, 0, 2).reshape(N_NODES, N_SUP * D_OUT)
    return (user_out, item_out)
